# baseline (speedup 1.0000x reference)
import functools

import jax
import jax.numpy as jnp
from jax import lax
from jax.experimental import pallas as pl
from jax.experimental.pallas import tpu as pltpu
from jax.experimental.pallas import tpu_sc as plsc

EMBED = 128
NC = 2
NS = 16
NW = NC * NS
CH = 128
RB = 4


def kernel(x, table):
    B, L = x.shape
    n = B * L
    per_w = n // NW
    nch = per_w // CH
    idx = x.reshape(NW, nch, CH).astype(jnp.int32)

    mesh = plsc.VectorSubcoreMesh(core_axis_name="c", subcore_axis_name="s")

    scratch = (
        [pltpu.VMEM((CH,), jnp.int32) for _ in range(RB)]
        + [pltpu.VMEM((CH, EMBED), jnp.float32) for _ in range(RB)]
        + [pltpu.SemaphoreType.DMA for _ in range(3 * RB)]
    )

    @functools.partial(
        pl.kernel,
        out_type=jax.ShapeDtypeStruct((n, EMBED), table.dtype),
        mesh=mesh,
        scratch_types=scratch,
    )
    def gather_kernel(table_hbm, idx_hbm, out_hbm, *refs):
        idxc = refs[0:RB]
        bufs = refs[RB:2 * RB]
        isem = refs[2 * RB:3 * RB]
        gsem = refs[3 * RB:4 * RB]
        wsem = refs[4 * RB:5 * RB]

        wid = lax.axis_index("s") * NC + lax.axis_index("c")
        base = wid * per_w

        def idx_start(i, b):
            pltpu.async_copy(idx_hbm.at[wid, i], idxc[b], isem[b])

        def idx_wait(i, b):
            pltpu.make_async_copy(idx_hbm.at[wid, i], idxc[b], isem[b]).wait()

        def gather_start(b):
            pltpu.async_copy(table_hbm.at[idxc[b]], bufs[b], gsem[b])

        def gather_wait(b):
            pltpu.make_async_copy(table_hbm.at[idxc[b]], bufs[b], gsem[b]).wait()

        def write_start(i, b):
            pltpu.async_copy(bufs[b], out_hbm.at[pl.ds(base + i * CH, CH)],
                             wsem[b])

        def write_wait(i, b):
            pltpu.make_async_copy(
                bufs[b], out_hbm.at[pl.ds(base + i * CH, CH)], wsem[b]
            ).wait()

        for j in range(RB - 1):
            pltpu.sync_copy(idx_hbm.at[wid, j], idxc[j])
            gather_start(j)
        idx_start(RB - 1, RB - 1)

        @pl.loop(0, nch // RB)
        def _(it):
            i0 = it * RB
            for r in range(RB):
                i = i0 + r
                b = r
                bp = (r - 1) % RB
                gather_wait(b)

                @pl.when(i + RB < nch)
                def _():
                    idx_start(i + RB, b)

                write_start(i, b)

                @pl.when(i > 0)
                def _():
                    write_wait(i - 1, bp)

                @pl.when(i + RB - 1 < nch)
                def _():
                    idx_wait(i + RB - 1, bp)
                    gather_start(bp)

        write_wait(nch - 1, (nch - 1) % RB)

    out = gather_kernel(table, idx)
    return out.reshape(B, L, EMBED)


# speedup vs baseline: 1.0002x; 1.0002x over previous
import functools

import jax
import jax.numpy as jnp
from jax import lax
from jax.experimental import pallas as pl
from jax.experimental.pallas import tpu as pltpu
from jax.experimental.pallas import tpu_sc as plsc

EMBED = 128
NC = 2
NS = 16
NW = NC * NS
CH = 256
RB = 2


def kernel(x, table):
    B, L = x.shape
    n = B * L
    per_w = n // NW
    nch = per_w // CH
    idx = x.reshape(NW, nch, CH).astype(jnp.int32)

    mesh = plsc.VectorSubcoreMesh(core_axis_name="c", subcore_axis_name="s")

    scratch = (
        [pltpu.VMEM((CH,), jnp.int32) for _ in range(RB)]
        + [pltpu.VMEM((CH, EMBED), jnp.float32) for _ in range(RB)]
        + [pltpu.SemaphoreType.DMA for _ in range(3 * RB)]
    )

    @functools.partial(
        pl.kernel,
        out_type=jax.ShapeDtypeStruct((n, EMBED), table.dtype),
        mesh=mesh,
        scratch_types=scratch,
    )
    def gather_kernel(table_hbm, idx_hbm, out_hbm, *refs):
        idxc = refs[0:RB]
        bufs = refs[RB:2 * RB]
        isem = refs[2 * RB:3 * RB]
        gsem = refs[3 * RB:4 * RB]
        wsem = refs[4 * RB:5 * RB]

        wid = lax.axis_index("s") * NC + lax.axis_index("c")
        base = wid * per_w

        def idx_start(i, b):
            pltpu.async_copy(idx_hbm.at[wid, i], idxc[b], isem[b])

        def idx_wait(i, b):
            pltpu.make_async_copy(idx_hbm.at[wid, i], idxc[b], isem[b]).wait()

        def gather_start(b):
            pltpu.async_copy(table_hbm.at[idxc[b]], bufs[b], gsem[b])

        def gather_wait(b):
            pltpu.make_async_copy(table_hbm.at[idxc[b]], bufs[b], gsem[b]).wait()

        def write_start(i, b):
            pltpu.async_copy(bufs[b], out_hbm.at[pl.ds(base + i * CH, CH)],
                             wsem[b])

        def write_wait(i, b):
            pltpu.make_async_copy(
                bufs[b], out_hbm.at[pl.ds(base + i * CH, CH)], wsem[b]
            ).wait()

        for j in range(RB - 1):
            pltpu.sync_copy(idx_hbm.at[wid, j], idxc[j])
            gather_start(j)
        idx_start(RB - 1, RB - 1)

        @pl.loop(0, nch // RB)
        def _(it):
            i0 = it * RB
            for r in range(RB):
                i = i0 + r
                b = r
                bp = (r - 1) % RB
                gather_wait(b)

                @pl.when(i + RB < nch)
                def _():
                    idx_start(i + RB, b)

                write_start(i, b)

                @pl.when(i > 0)
                def _():
                    write_wait(i - 1, bp)

                @pl.when(i + RB - 1 < nch)
                def _():
                    idx_wait(i + RB - 1, bp)
                    gather_start(bp)

        write_wait(nch - 1, (nch - 1) % RB)

    out = gather_kernel(table, idx)
    return out.reshape(B, L, EMBED)
